# baseline (speedup 1.0000x reference)
import functools

import jax
import jax.numpy as jnp
from jax import lax
from jax.experimental import pallas as pl
from jax.experimental.pallas import tpu as pltpu
from jax.experimental.pallas import tpu_sc as plsc

N = 10000
E = 320000

NC = 2
NS = 16
CHUNK = 128
KI = 32
NBLK = 5
NCH = KI * NBLK
EP = NCH * CHUNK
EPAD = NS * EP
TRASH = N
RPT = 640
R = NS * RPT
NOUT = R


def _make_seg_sum(dh):
  mesh = plsc.VectorSubcoreMesh(core_axis_name="c", subcore_axis_name="s",
                                num_cores=NC, num_subcores=NS)

  @functools.partial(
      pl.kernel,
      out_type=jax.ShapeDtypeStruct((NC, NOUT, dh), jnp.float32),
      mesh=mesh,
      scratch_types=[
          pltpu.VMEM((KI, CHUNK), jnp.int32),
          pltpu.VMEM((KI, CHUNK), jnp.int32),
          pltpu.VMEM((CHUNK, dh), jnp.float32),
          pltpu.VMEM((CHUNK, dh), jnp.float32),
          pltpu.VMEM_SHARED((R, dh), jnp.float32),
          pltpu.SemaphoreType.DMA,
          pltpu.SemaphoreType.DMA,
          pltpu.SemaphoreType.DMA,
          pltpu.SemaphoreType.DMA,
      ],
  )
  def seg_sum(y_hbm, srcs_hbm, dsts_hbm, zeros_hbm, out_hbm,
              src_v, dst_v, rows0, rows1, acc_sh, ga, gb, sa, sb):
    c = lax.axis_index("c")
    s = lax.axis_index("s")
    pltpu.sync_copy(zeros_hbm, acc_sh.at[pl.ds(s * RPT, RPT)])
    plsc.subcore_barrier()
    table = y_hbm.at[c]

    def blk(bi, carry):
      pltpu.sync_copy(srcs_hbm.at[s].at[pl.ds(bi * KI, KI)], src_v)
      pltpu.sync_copy(dsts_hbm.at[s].at[pl.ds(bi * KI, KI)], dst_v)
      pltpu.async_copy(table.at[src_v.at[0]], rows0, ga)
      pltpu.async_copy(table.at[src_v.at[1]], rows1, gb)

      def pair(k, carry2):
        j0 = 2 * k
        pltpu.make_async_copy(table.at[src_v.at[0]], rows0, ga).wait()
        pltpu.async_copy(rows0, acc_sh.at[dst_v.at[j0]], sa, add=True)
        pltpu.make_async_copy(table.at[src_v.at[0]], rows1, gb).wait()
        pltpu.async_copy(rows1, acc_sh.at[dst_v.at[j0 + 1]], sb, add=True)
        pltpu.make_async_copy(rows0, acc_sh.at[dst_v.at[0]], sa).wait()

        @pl.when(j0 + 2 < KI)
        def _():
          pltpu.async_copy(table.at[src_v.at[j0 + 2]], rows0, ga)

        pltpu.make_async_copy(rows1, acc_sh.at[dst_v.at[0]], sb).wait()

        @pl.when(j0 + 3 < KI)
        def _():
          pltpu.async_copy(table.at[src_v.at[j0 + 3]], rows1, gb)

        return carry2

      return lax.fori_loop(0, KI // 2, pair, carry)

    lax.fori_loop(0, NBLK, blk, 0)
    plsc.subcore_barrier()
    pltpu.sync_copy(acc_sh.at[pl.ds(s * RPT, RPT)],
                    out_hbm.at[c].at[pl.ds(s * RPT, RPT)])

  return seg_sum


_seg_sum_128 = _make_seg_sum(128)

_deg_mesh = plsc.VectorSubcoreMesh(core_axis_name="c", subcore_axis_name="s",
                                   num_cores=NC, num_subcores=NS)


@functools.partial(
    pl.kernel,
    out_type=jax.ShapeDtypeStruct((NC, NOUT, 128), jnp.float32),
    mesh=_deg_mesh,
    scratch_types=[
        pltpu.VMEM((KI, CHUNK), jnp.int32),
        pltpu.VMEM((CHUNK, 128), jnp.float32),
        pltpu.VMEM_SHARED((R, 128), jnp.float32),
        pltpu.SemaphoreType.DMA,
    ],
)
def _deg_kernel(dsts_hbm, ones_hbm, zeros_hbm, out_hbm,
                dst_v, rows_v, acc_sh, deg_sem):
  s = lax.axis_index("s")
  pltpu.sync_copy(ones_hbm, rows_v)
  pltpu.sync_copy(zeros_hbm, acc_sh.at[pl.ds(s * RPT, RPT)])
  plsc.subcore_barrier()

  def blk(bi, carry):
    pltpu.sync_copy(dsts_hbm.at[s].at[pl.ds(bi * KI, KI)], dst_v)

    def fire(j, carry2):
      pltpu.async_copy(rows_v, acc_sh.at[dst_v.at[j]], deg_sem, add=True)
      return carry2

    lax.fori_loop(0, KI, fire, carry)

    def drain(j, carry2):
      pltpu.make_async_copy(rows_v, acc_sh.at[dst_v.at[0]], deg_sem).wait()
      return carry2

    return lax.fori_loop(0, KI, drain, carry)

  lax.fori_loop(0, NBLK, blk, 0)
  plsc.subcore_barrier()
  pltpu.sync_copy(acc_sh.at[pl.ds(s * RPT, RPT)],
                  out_hbm.at[lax.axis_index("c")].at[pl.ds(s * RPT, RPT)])


B = 1000
_GRID = N // B


def _pre0_body(deg_ref, x_ref, w_ref, dinv_ref, y_ref):
  dinv = lax.rsqrt(deg_ref[0, :, 0:1] + 1.0)
  xw = jnp.dot(x_ref[...], w_ref[...], preferred_element_type=jnp.float32)
  y = xw * dinv
  dh = y.shape[-1] // 2
  dinv_ref[...] = dinv
  y_ref[0] = y[:, :dh]
  y_ref[1] = y[:, dh:]


def _pre_body(dinv_ref, h_ref, w_ref, y_ref):
  xw = jnp.dot(h_ref[...], w_ref[...], preferred_element_type=jnp.float32)
  y = xw * dinv_ref[...]
  dh = y.shape[-1] // 2
  y_ref[0] = y[:, :dh]
  y_ref[1] = y[:, dh:]


def _pre3_body(dinv_ref, h_ref, w_ref, y_ref):
  xw = jnp.dot(h_ref[...], w_ref[...], preferred_element_type=jnp.float32)
  y_ref[0] = xw * dinv_ref[...]
  y_ref[1] = jnp.zeros_like(xw)


def _post_body(a_ref, y_ref, dinv_ref, res_ref, b_ref, g_ref, be_ref, o_ref):
  agg = jnp.concatenate([a_ref[0], a_ref[1]], axis=-1)
  y = jnp.concatenate([y_ref[0], y_ref[1]], axis=-1)
  dinv = dinv_ref[...]
  conv = dinv * (agg + y) + b_ref[...]
  mu = jnp.mean(conv, axis=-1, keepdims=True)
  var = jnp.mean((conv - mu) ** 2, axis=-1, keepdims=True)
  xn = (conv - mu) * lax.rsqrt(var + 1e-5) * g_ref[...] + be_ref[...]
  o = jnp.where(xn > 0, xn, jnp.exp(jnp.minimum(xn, 0.0)) - 1.0)
  o_ref[...] = o + res_ref[...]


def _final_body(a_ref, y_ref, dinv_ref, b_ref, o_ref):
  o_ref[...] = dinv_ref[...] * (a_ref[0] + y_ref[0]) + b_ref[...]


def _row_spec(d):
  return pl.BlockSpec((B, d), lambda i: (i, 0))


def _pair_spec(dh):
  return pl.BlockSpec((2, B, dh), lambda i: (0, i, 0))


def _full_spec(shape):
  return pl.BlockSpec(shape, lambda i: tuple(0 for _ in shape))


def _pre0_call(degcol, x, w):
  dout = w.shape[1]
  return pl.pallas_call(
      _pre0_body,
      grid=(_GRID,),
      in_specs=[pl.BlockSpec((1, B, 128), lambda i: (0, i, 0)),
                _row_spec(x.shape[1]), _full_spec(w.shape)],
      out_specs=[_row_spec(1), _pair_spec(dout // 2)],
      out_shape=[
          jax.ShapeDtypeStruct((N, 1), jnp.float32),
          jax.ShapeDtypeStruct((2, N, dout // 2), jnp.float32),
      ],
  )(degcol, x, w)


def _pre_call(dinv, h, w):
  dout = w.shape[1]
  return pl.pallas_call(
      _pre_body,
      grid=(_GRID,),
      in_specs=[_row_spec(1), _row_spec(h.shape[1]), _full_spec(w.shape)],
      out_specs=_pair_spec(dout // 2),
      out_shape=jax.ShapeDtypeStruct((2, N, dout // 2), jnp.float32),
  )(dinv, h, w)


def _post_call(agg, y, dinv, res, b, g, be):
  dh = agg.shape[-1]
  d = 2 * dh
  return pl.pallas_call(
      _post_body,
      grid=(_GRID,),
      in_specs=[_pair_spec(dh), _pair_spec(dh), _row_spec(1), _row_spec(d),
                _full_spec((1, d)), _full_spec((1, d)), _full_spec((1, d))],
      out_specs=_row_spec(d),
      out_shape=jax.ShapeDtypeStruct((N, d), jnp.float32),
  )(agg, y, dinv, res, b.reshape(1, d), g.reshape(1, d), be.reshape(1, d))


def _pre3_call(dinv, h, w):
  dout = w.shape[1]
  return pl.pallas_call(
      _pre3_body,
      grid=(_GRID,),
      in_specs=[_row_spec(1), _row_spec(h.shape[1]), _full_spec(w.shape)],
      out_specs=_pair_spec(dout),
      out_shape=jax.ShapeDtypeStruct((2, N, dout), jnp.float32),
  )(dinv, h, w)


def _final_call(agg, y, dinv, b):
  d = agg.shape[-1]
  return pl.pallas_call(
      _final_body,
      grid=(_GRID,),
      in_specs=[_pair_spec(d), _pair_spec(d), _row_spec(1),
                _full_spec((1, d))],
      out_specs=_row_spec(d),
      out_shape=jax.ShapeDtypeStruct((N, d), jnp.float32),
  )(agg, y, dinv, b.reshape(1, d))


def kernel(x, edge_index, W0, b0, W1, b1, W2, b2, W3, b3,
           g0, be0, g1, be1, g2, be2):
  src = edge_index[0].astype(jnp.int32)
  dst = edge_index[1].astype(jnp.int32)
  pad = EPAD - E
  srcs = jnp.pad(src, (0, pad)).reshape(NS, NCH, CHUNK)
  dsts = jnp.pad(dst, (0, pad), constant_values=TRASH).reshape(NS, NCH, CHUNK)
  zeros128 = jnp.zeros((RPT, 128), jnp.float32)
  ones128 = jnp.ones((CHUNK, 128), jnp.float32)

  deg_out = _deg_kernel(dsts, ones128, zeros128)

  dinv, y = _pre0_call(deg_out, x, W0)
  agg = _seg_sum_128(y, srcs, dsts, zeros128)
  h = _post_call(agg, y, dinv, jnp.zeros((N, 256), jnp.float32), b0, g0, be0)

  for (w, b, g, be) in ((W1, b1, g1, be1), (W2, b2, g2, be2)):
    y = _pre_call(dinv, h, w)
    agg = _seg_sum_128(y, srcs, dsts, zeros128)
    h = _post_call(agg, y, dinv, h, b, g, be)

  y = _pre3_call(dinv, h, W3)
  agg = _seg_sum_128(y, srcs, dsts, zeros128)
  return _final_call(agg, y, dinv, b3)


# speedup vs baseline: 1.0168x; 1.0168x over previous
import functools

import jax
import jax.numpy as jnp
from jax import lax
from jax.experimental import pallas as pl
from jax.experimental.pallas import tpu as pltpu
from jax.experimental.pallas import tpu_sc as plsc

N = 10000
E = 320000

NC = 2
NS = 16
CHUNK = 128
KI = 32
NBLK = 5
NCH = KI * NBLK
EP = NCH * CHUNK
EPAD = NS * EP
TRASH = N
RPT = 640
R = NS * RPT
NOUT = R


def _make_seg_sum(dh):
  mesh = plsc.VectorSubcoreMesh(core_axis_name="c", subcore_axis_name="s",
                                num_cores=NC, num_subcores=NS)

  @functools.partial(
      pl.kernel,
      out_type=jax.ShapeDtypeStruct((NC, NOUT, dh), jnp.float32),
      mesh=mesh,
      scratch_types=[
          pltpu.VMEM((KI, CHUNK), jnp.int32),
          pltpu.VMEM((KI, CHUNK), jnp.int32),
          pltpu.VMEM((CHUNK, dh), jnp.float32),
          pltpu.VMEM((CHUNK, dh), jnp.float32),
          pltpu.VMEM_SHARED((R, dh), jnp.float32),
          pltpu.SemaphoreType.DMA,
          pltpu.SemaphoreType.DMA,
      ],
  )
  def seg_sum(y_hbm, srcs_hbm, dsts_hbm, zeros_hbm, out_hbm,
              src_v, dst_v, rows0, rows1, acc_sh, sem0, sem1):
    c = lax.axis_index("c")
    s = lax.axis_index("s")
    pltpu.sync_copy(zeros_hbm, acc_sh.at[pl.ds(s * RPT, RPT)])
    plsc.subcore_barrier()
    table = y_hbm.at[c]

    def blk(bi, carry):
      pltpu.sync_copy(srcs_hbm.at[s].at[pl.ds(bi * KI, KI)], src_v)
      pltpu.sync_copy(dsts_hbm.at[s].at[pl.ds(bi * KI, KI)], dst_v)
      pltpu.async_copy(table.at[src_v.at[0]], rows0, sem0)

      def pair(k, carry2):
        j0 = 2 * k
        pltpu.make_async_copy(table.at[src_v.at[0]], rows0, sem0).wait()
        pltpu.async_copy(table.at[src_v.at[j0 + 1]], rows1, sem1)
        pltpu.sync_copy(rows0, acc_sh.at[dst_v.at[j0]], add=True)
        pltpu.make_async_copy(table.at[src_v.at[0]], rows1, sem1).wait()

        @pl.when(j0 + 2 < KI)
        def _():
          pltpu.async_copy(table.at[src_v.at[j0 + 2]], rows0, sem0)

        pltpu.sync_copy(rows1, acc_sh.at[dst_v.at[j0 + 1]], add=True)
        return carry2

      return lax.fori_loop(0, KI // 2, pair, carry)

    lax.fori_loop(0, NBLK, blk, 0)
    plsc.subcore_barrier()
    pltpu.sync_copy(acc_sh.at[pl.ds(s * RPT, RPT)],
                    out_hbm.at[c].at[pl.ds(s * RPT, RPT)])

  return seg_sum


_seg_sum_128 = _make_seg_sum(128)

_deg_mesh = plsc.VectorSubcoreMesh(core_axis_name="c", subcore_axis_name="s",
                                   num_cores=NC, num_subcores=NS)


@functools.partial(
    pl.kernel,
    out_type=jax.ShapeDtypeStruct((NC, NOUT, 128), jnp.float32),
    mesh=_deg_mesh,
    scratch_types=[
        pltpu.VMEM((KI, CHUNK), jnp.int32),
        pltpu.VMEM((CHUNK, 128), jnp.float32),
        pltpu.VMEM_SHARED((R, 128), jnp.float32),
        pltpu.SemaphoreType.DMA,
    ],
)
def _deg_kernel(dsts_hbm, ones_hbm, zeros_hbm, out_hbm,
                dst_v, rows_v, acc_sh, deg_sem):
  s = lax.axis_index("s")
  pltpu.sync_copy(ones_hbm, rows_v)
  pltpu.sync_copy(zeros_hbm, acc_sh.at[pl.ds(s * RPT, RPT)])
  plsc.subcore_barrier()

  def blk(bi, carry):
    pltpu.sync_copy(dsts_hbm.at[s].at[pl.ds(bi * KI, KI)], dst_v)

    def fire(j, carry2):
      pltpu.async_copy(rows_v, acc_sh.at[dst_v.at[j]], deg_sem, add=True)
      return carry2

    lax.fori_loop(0, KI, fire, carry)

    def drain(j, carry2):
      pltpu.make_async_copy(rows_v, acc_sh.at[dst_v.at[0]], deg_sem).wait()
      return carry2

    return lax.fori_loop(0, KI, drain, carry)

  lax.fori_loop(0, NBLK, blk, 0)
  plsc.subcore_barrier()
  pltpu.sync_copy(acc_sh.at[pl.ds(s * RPT, RPT)],
                  out_hbm.at[lax.axis_index("c")].at[pl.ds(s * RPT, RPT)])


B = 1000
_GRID = N // B


def _pre0_body(deg_ref, x_ref, w_ref, dinv_ref, y_ref):
  dinv = lax.rsqrt(deg_ref[0, :, 0:1] + 1.0)
  xw = jnp.dot(x_ref[...], w_ref[...], preferred_element_type=jnp.float32)
  y = xw * dinv
  dh = y.shape[-1] // 2
  dinv_ref[...] = dinv
  y_ref[0] = y[:, :dh]
  y_ref[1] = y[:, dh:]


def _pre_body(dinv_ref, h_ref, w_ref, y_ref):
  xw = jnp.dot(h_ref[...], w_ref[...], preferred_element_type=jnp.float32)
  y = xw * dinv_ref[...]
  dh = y.shape[-1] // 2
  y_ref[0] = y[:, :dh]
  y_ref[1] = y[:, dh:]


def _pre3_body(dinv_ref, h_ref, w_ref, y_ref):
  xw = jnp.dot(h_ref[...], w_ref[...], preferred_element_type=jnp.float32)
  y_ref[0] = xw * dinv_ref[...]
  y_ref[1] = jnp.zeros_like(xw)


def _post_body(a_ref, y_ref, dinv_ref, res_ref, b_ref, g_ref, be_ref, o_ref):
  agg = jnp.concatenate([a_ref[0], a_ref[1]], axis=-1)
  y = jnp.concatenate([y_ref[0], y_ref[1]], axis=-1)
  dinv = dinv_ref[...]
  conv = dinv * (agg + y) + b_ref[...]
  mu = jnp.mean(conv, axis=-1, keepdims=True)
  var = jnp.mean((conv - mu) ** 2, axis=-1, keepdims=True)
  xn = (conv - mu) * lax.rsqrt(var + 1e-5) * g_ref[...] + be_ref[...]
  o = jnp.where(xn > 0, xn, jnp.exp(jnp.minimum(xn, 0.0)) - 1.0)
  o_ref[...] = o + res_ref[...]


def _final_body(a_ref, y_ref, dinv_ref, b_ref, o_ref):
  o_ref[...] = dinv_ref[...] * (a_ref[0] + y_ref[0]) + b_ref[...]


def _row_spec(d):
  return pl.BlockSpec((B, d), lambda i: (i, 0))


def _pair_spec(dh):
  return pl.BlockSpec((2, B, dh), lambda i: (0, i, 0))


def _full_spec(shape):
  return pl.BlockSpec(shape, lambda i: tuple(0 for _ in shape))


def _pre0_call(degcol, x, w):
  dout = w.shape[1]
  return pl.pallas_call(
      _pre0_body,
      grid=(_GRID,),
      in_specs=[pl.BlockSpec((1, B, 128), lambda i: (0, i, 0)),
                _row_spec(x.shape[1]), _full_spec(w.shape)],
      out_specs=[_row_spec(1), _pair_spec(dout // 2)],
      out_shape=[
          jax.ShapeDtypeStruct((N, 1), jnp.float32),
          jax.ShapeDtypeStruct((2, N, dout // 2), jnp.float32),
      ],
  )(degcol, x, w)


def _pre_call(dinv, h, w):
  dout = w.shape[1]
  return pl.pallas_call(
      _pre_body,
      grid=(_GRID,),
      in_specs=[_row_spec(1), _row_spec(h.shape[1]), _full_spec(w.shape)],
      out_specs=_pair_spec(dout // 2),
      out_shape=jax.ShapeDtypeStruct((2, N, dout // 2), jnp.float32),
  )(dinv, h, w)


def _post_call(agg, y, dinv, res, b, g, be):
  dh = agg.shape[-1]
  d = 2 * dh
  return pl.pallas_call(
      _post_body,
      grid=(_GRID,),
      in_specs=[_pair_spec(dh), _pair_spec(dh), _row_spec(1), _row_spec(d),
                _full_spec((1, d)), _full_spec((1, d)), _full_spec((1, d))],
      out_specs=_row_spec(d),
      out_shape=jax.ShapeDtypeStruct((N, d), jnp.float32),
  )(agg, y, dinv, res, b.reshape(1, d), g.reshape(1, d), be.reshape(1, d))


def _pre3_call(dinv, h, w):
  dout = w.shape[1]
  return pl.pallas_call(
      _pre3_body,
      grid=(_GRID,),
      in_specs=[_row_spec(1), _row_spec(h.shape[1]), _full_spec(w.shape)],
      out_specs=_pair_spec(dout),
      out_shape=jax.ShapeDtypeStruct((2, N, dout), jnp.float32),
  )(dinv, h, w)


def _final_call(agg, y, dinv, b):
  d = agg.shape[-1]
  return pl.pallas_call(
      _final_body,
      grid=(_GRID,),
      in_specs=[_pair_spec(d), _pair_spec(d), _row_spec(1),
                _full_spec((1, d))],
      out_specs=_row_spec(d),
      out_shape=jax.ShapeDtypeStruct((N, d), jnp.float32),
  )(agg, y, dinv, b.reshape(1, d))


def kernel(x, edge_index, W0, b0, W1, b1, W2, b2, W3, b3,
           g0, be0, g1, be1, g2, be2):
  src = edge_index[0].astype(jnp.int32)
  dst = edge_index[1].astype(jnp.int32)
  pad = EPAD - E
  srcs = jnp.pad(src, (0, pad)).reshape(NS, NCH, CHUNK)
  dsts = jnp.pad(dst, (0, pad), constant_values=TRASH).reshape(NS, NCH, CHUNK)
  zeros128 = jnp.zeros((RPT, 128), jnp.float32)
  ones128 = jnp.ones((CHUNK, 128), jnp.float32)

  deg_out = _deg_kernel(dsts, ones128, zeros128)

  dinv, y = _pre0_call(deg_out, x, W0)
  agg = _seg_sum_128(y, srcs, dsts, zeros128)
  h = _post_call(agg, y, dinv, jnp.zeros((N, 256), jnp.float32), b0, g0, be0)

  for (w, b, g, be) in ((W1, b1, g1, be1), (W2, b2, g2, be2)):
    y = _pre_call(dinv, h, w)
    agg = _seg_sum_128(y, srcs, dsts, zeros128)
    h = _post_call(agg, y, dinv, h, b, g, be)

  y = _pre3_call(dinv, h, W3)
  agg = _seg_sum_128(y, srcs, dsts, zeros128)
  return _final_call(agg, y, dinv, b3)


# speedup vs baseline: 1.0267x; 1.0097x over previous
import functools

import jax
import jax.numpy as jnp
from jax import lax
from jax.experimental import pallas as pl
from jax.experimental.pallas import tpu as pltpu
from jax.experimental.pallas import tpu_sc as plsc

N = 10000
E = 320000

NC = 2
NS = 16
CHUNK = 128
KI = 32
NBLK = 5
NCH = KI * NBLK
EP = NCH * CHUNK
EPAD = NS * EP
TRASH = N
RPT = 640
R = NS * RPT
NOUT = R


def _make_seg_sum(dh):
  mesh = plsc.VectorSubcoreMesh(core_axis_name="c", subcore_axis_name="s",
                                num_cores=NC, num_subcores=NS)

  @functools.partial(
      pl.kernel,
      out_type=jax.ShapeDtypeStruct((NC, NOUT, dh), jnp.float32),
      mesh=mesh,
      scratch_types=[
          pltpu.VMEM((KI, CHUNK), jnp.int32),
          pltpu.VMEM((KI, CHUNK), jnp.int32),
          pltpu.VMEM((CHUNK, dh), jnp.float32),
          pltpu.VMEM((CHUNK, dh), jnp.float32),
          pltpu.VMEM_SHARED((R, dh), jnp.float32),
          pltpu.SemaphoreType.DMA,
          pltpu.SemaphoreType.DMA,
      ],
  )
  def seg_sum(y_hbm, srcs_hbm, dsts_hbm, zeros_hbm, out_hbm,
              src_v, dst_v, rows0, rows1, acc_sh, sem0, sem1):
    c = lax.axis_index("c")
    s = lax.axis_index("s")
    pltpu.sync_copy(zeros_hbm, acc_sh.at[pl.ds(s * RPT, RPT)])
    plsc.subcore_barrier()
    table = y_hbm.at[c]

    def blk(bi, carry):
      pltpu.sync_copy(srcs_hbm.at[s].at[pl.ds(bi * KI, KI)], src_v)
      pltpu.sync_copy(dsts_hbm.at[s].at[pl.ds(bi * KI, KI)], dst_v)
      pltpu.async_copy(table.at[src_v.at[0]], rows0, sem0)

      def pair(k, carry2):
        j0 = 2 * k
        pltpu.make_async_copy(table.at[src_v.at[0]], rows0, sem0).wait()
        pltpu.async_copy(table.at[src_v.at[j0 + 1]], rows1, sem1)
        pltpu.sync_copy(rows0, acc_sh.at[dst_v.at[j0]], add=True)
        pltpu.make_async_copy(table.at[src_v.at[0]], rows1, sem1).wait()

        @pl.when(j0 + 2 < KI)
        def _():
          pltpu.async_copy(table.at[src_v.at[j0 + 2]], rows0, sem0)

        pltpu.sync_copy(rows1, acc_sh.at[dst_v.at[j0 + 1]], add=True)
        return carry2

      return lax.fori_loop(0, KI // 2, pair, carry)

    lax.fori_loop(0, NBLK, blk, 0)
    plsc.subcore_barrier()
    pltpu.sync_copy(acc_sh.at[pl.ds(s * RPT, RPT)],
                    out_hbm.at[c].at[pl.ds(s * RPT, RPT)])

  return seg_sum


_seg_sum_128 = _make_seg_sum(128)

_deg_mesh = plsc.VectorSubcoreMesh(core_axis_name="c", subcore_axis_name="s",
                                   num_cores=NC, num_subcores=NS)


@functools.partial(
    pl.kernel,
    out_type=jax.ShapeDtypeStruct((NC, NOUT, 128), jnp.float32),
    mesh=_deg_mesh,
    scratch_types=[
        pltpu.VMEM((KI, CHUNK), jnp.int32),
        pltpu.VMEM((CHUNK, 128), jnp.float32),
        pltpu.VMEM_SHARED((R, 128), jnp.float32),
        pltpu.SemaphoreType.DMA,
    ],
)
def _deg_kernel(dsts_hbm, ones_hbm, zeros_hbm, out_hbm,
                dst_v, rows_v, acc_sh, deg_sem):
  s = lax.axis_index("s")
  pltpu.sync_copy(ones_hbm, rows_v)
  pltpu.sync_copy(zeros_hbm, acc_sh.at[pl.ds(s * RPT, RPT)])
  plsc.subcore_barrier()

  def blk(bi, carry):
    pltpu.sync_copy(dsts_hbm.at[s].at[pl.ds(bi * KI, KI)], dst_v)

    def fire(j, carry2):
      pltpu.async_copy(rows_v, acc_sh.at[dst_v.at[j]], deg_sem, add=True)
      return carry2

    lax.fori_loop(0, KI, fire, carry)

    def drain(j, carry2):
      pltpu.make_async_copy(rows_v, acc_sh.at[dst_v.at[0]], deg_sem).wait()
      return carry2

    return lax.fori_loop(0, KI, drain, carry)

  lax.fori_loop(0, NBLK, blk, 0)
  plsc.subcore_barrier()
  pltpu.sync_copy(acc_sh.at[pl.ds(s * RPT, RPT)],
                  out_hbm.at[lax.axis_index("c")].at[pl.ds(s * RPT, RPT)])


B = 1000
_GRID = N // B


def _pre0_body(deg_ref, x_ref, w_ref, dinv_ref, y_ref):
  dinv = lax.rsqrt(deg_ref[0, :, 0:1] + 1.0)
  xw = jnp.dot(x_ref[...], w_ref[...], preferred_element_type=jnp.float32)
  y = xw * dinv
  dh = y.shape[-1] // 2
  dinv_ref[...] = dinv
  y_ref[0] = y[:, :dh]
  y_ref[1] = y[:, dh:]


def _pre_body(dinv_ref, h_ref, w_ref, y_ref):
  xw = jnp.dot(h_ref[...], w_ref[...], preferred_element_type=jnp.float32)
  y = xw * dinv_ref[...]
  dh = y.shape[-1] // 2
  y_ref[0] = y[:, :dh]
  y_ref[1] = y[:, dh:]


def _pre3_body(dinv_ref, h_ref, w_ref, y_ref):
  xw = jnp.dot(h_ref[...], w_ref[...], preferred_element_type=jnp.float32)
  y_ref[0] = xw * dinv_ref[...]
  y_ref[1] = jnp.zeros_like(xw)


def _post_body(a_ref, y_ref, dinv_ref, res_ref, b_ref, g_ref, be_ref, o_ref):
  agg = jnp.concatenate([a_ref[0], a_ref[1]], axis=-1)
  y = jnp.concatenate([y_ref[0], y_ref[1]], axis=-1)
  dinv = dinv_ref[...]
  conv = dinv * (agg + y) + b_ref[...]
  mu = jnp.mean(conv, axis=-1, keepdims=True)
  var = jnp.mean((conv - mu) ** 2, axis=-1, keepdims=True)
  xn = (conv - mu) * lax.rsqrt(var + 1e-5) * g_ref[...] + be_ref[...]
  o = jnp.where(xn > 0, xn, jnp.exp(jnp.minimum(xn, 0.0)) - 1.0)
  o_ref[...] = o + res_ref[...]


def _post_pre_body(a_ref, y_ref, dinv_ref, res_ref, b_ref, g_ref, be_ref,
                   w_ref, h_ref, yn_ref):
  agg = jnp.concatenate([a_ref[0], a_ref[1]], axis=-1)
  y = jnp.concatenate([y_ref[0], y_ref[1]], axis=-1)
  dinv = dinv_ref[...]
  conv = dinv * (agg + y) + b_ref[...]
  mu = jnp.mean(conv, axis=-1, keepdims=True)
  var = jnp.mean((conv - mu) ** 2, axis=-1, keepdims=True)
  xn = (conv - mu) * lax.rsqrt(var + 1e-5) * g_ref[...] + be_ref[...]
  o = jnp.where(xn > 0, xn, jnp.exp(jnp.minimum(xn, 0.0)) - 1.0)
  h = o + res_ref[...]
  h_ref[...] = h
  xw = jnp.dot(h, w_ref[...], preferred_element_type=jnp.float32)
  yn = xw * dinv
  dh = yn.shape[-1] // 2
  yn_ref[0] = yn[:, :dh]
  yn_ref[1] = yn[:, dh:]


def _post_pre3_body(a_ref, y_ref, dinv_ref, res_ref, b_ref, g_ref, be_ref,
                    w_ref, h_ref, yn_ref):
  agg = jnp.concatenate([a_ref[0], a_ref[1]], axis=-1)
  y = jnp.concatenate([y_ref[0], y_ref[1]], axis=-1)
  dinv = dinv_ref[...]
  conv = dinv * (agg + y) + b_ref[...]
  mu = jnp.mean(conv, axis=-1, keepdims=True)
  var = jnp.mean((conv - mu) ** 2, axis=-1, keepdims=True)
  xn = (conv - mu) * lax.rsqrt(var + 1e-5) * g_ref[...] + be_ref[...]
  o = jnp.where(xn > 0, xn, jnp.exp(jnp.minimum(xn, 0.0)) - 1.0)
  h = o + res_ref[...]
  h_ref[...] = h
  xw = jnp.dot(h, w_ref[...], preferred_element_type=jnp.float32)
  yn_ref[0] = xw * dinv
  yn_ref[1] = jnp.zeros_like(xw)


def _final_body(a_ref, y_ref, dinv_ref, b_ref, o_ref):
  o_ref[...] = dinv_ref[...] * (a_ref[0] + y_ref[0]) + b_ref[...]


def _row_spec(d):
  return pl.BlockSpec((B, d), lambda i: (i, 0))


def _pair_spec(dh):
  return pl.BlockSpec((2, B, dh), lambda i: (0, i, 0))


def _full_spec(shape):
  return pl.BlockSpec(shape, lambda i: tuple(0 for _ in shape))


def _pre0_call(degcol, x, w):
  dout = w.shape[1]
  return pl.pallas_call(
      _pre0_body,
      grid=(_GRID,),
      in_specs=[pl.BlockSpec((1, B, 128), lambda i: (0, i, 0)),
                _row_spec(x.shape[1]), _full_spec(w.shape)],
      out_specs=[_row_spec(1), _pair_spec(dout // 2)],
      out_shape=[
          jax.ShapeDtypeStruct((N, 1), jnp.float32),
          jax.ShapeDtypeStruct((2, N, dout // 2), jnp.float32),
      ],
  )(degcol, x, w)


def _pre_call(dinv, h, w):
  dout = w.shape[1]
  return pl.pallas_call(
      _pre_body,
      grid=(_GRID,),
      in_specs=[_row_spec(1), _row_spec(h.shape[1]), _full_spec(w.shape)],
      out_specs=_pair_spec(dout // 2),
      out_shape=jax.ShapeDtypeStruct((2, N, dout // 2), jnp.float32),
  )(dinv, h, w)


def _post_call(agg, y, dinv, res, b, g, be):
  dh = agg.shape[-1]
  d = 2 * dh
  return pl.pallas_call(
      _post_body,
      grid=(_GRID,),
      in_specs=[_pair_spec(dh), _pair_spec(dh), _row_spec(1), _row_spec(d),
                _full_spec((1, d)), _full_spec((1, d)), _full_spec((1, d))],
      out_specs=_row_spec(d),
      out_shape=jax.ShapeDtypeStruct((N, d), jnp.float32),
  )(agg, y, dinv, res, b.reshape(1, d), g.reshape(1, d), be.reshape(1, d))


def _post_pre_call(agg, y, dinv, res, b, g, be, w, final):
  dh = agg.shape[-1]
  d = 2 * dh
  dout = w.shape[1]
  body = _post_pre3_body if final else _post_pre_body
  ydh = dout if final else dout // 2
  return pl.pallas_call(
      body,
      grid=(_GRID,),
      in_specs=[_pair_spec(dh), _pair_spec(dh), _row_spec(1), _row_spec(d),
                _full_spec((1, d)), _full_spec((1, d)), _full_spec((1, d)),
                _full_spec(w.shape)],
      out_specs=[_row_spec(d), _pair_spec(ydh)],
      out_shape=[
          jax.ShapeDtypeStruct((N, d), jnp.float32),
          jax.ShapeDtypeStruct((2, N, ydh), jnp.float32),
      ],
  )(agg, y, dinv, res, b.reshape(1, d), g.reshape(1, d), be.reshape(1, d), w)


def _pre3_call(dinv, h, w):
  dout = w.shape[1]
  return pl.pallas_call(
      _pre3_body,
      grid=(_GRID,),
      in_specs=[_row_spec(1), _row_spec(h.shape[1]), _full_spec(w.shape)],
      out_specs=_pair_spec(dout),
      out_shape=jax.ShapeDtypeStruct((2, N, dout), jnp.float32),
  )(dinv, h, w)


def _final_call(agg, y, dinv, b):
  d = agg.shape[-1]
  return pl.pallas_call(
      _final_body,
      grid=(_GRID,),
      in_specs=[_pair_spec(d), _pair_spec(d), _row_spec(1),
                _full_spec((1, d))],
      out_specs=_row_spec(d),
      out_shape=jax.ShapeDtypeStruct((N, d), jnp.float32),
  )(agg, y, dinv, b.reshape(1, d))


def kernel(x, edge_index, W0, b0, W1, b1, W2, b2, W3, b3,
           g0, be0, g1, be1, g2, be2):
  src = edge_index[0].astype(jnp.int32)
  dst = edge_index[1].astype(jnp.int32)
  pad = EPAD - E
  srcs = jnp.pad(src, (0, pad)).reshape(NS, NCH, CHUNK)
  dsts = jnp.pad(dst, (0, pad), constant_values=TRASH).reshape(NS, NCH, CHUNK)
  zeros128 = jnp.zeros((RPT, 128), jnp.float32)
  ones128 = jnp.ones((CHUNK, 128), jnp.float32)

  deg_out = _deg_kernel(dsts, ones128, zeros128)

  dinv, y = _pre0_call(deg_out, x, W0)
  agg = _seg_sum_128(y, srcs, dsts, zeros128)
  h, y = _post_pre_call(agg, y, dinv, jnp.zeros((N, 256), jnp.float32),
                        b0, g0, be0, W1, final=False)
  agg = _seg_sum_128(y, srcs, dsts, zeros128)
  h, y = _post_pre_call(agg, y, dinv, h, b1, g1, be1, W2, final=False)
  agg = _seg_sum_128(y, srcs, dsts, zeros128)
  h, y = _post_pre_call(agg, y, dinv, h, b2, g2, be2, W3, final=True)
  agg = _seg_sum_128(y, srcs, dsts, zeros128)
  return _final_call(agg, y, dinv, b3)


# speedup vs baseline: 1.1017x; 1.0730x over previous
import functools

import jax
import jax.numpy as jnp
from jax import lax
from jax.experimental import pallas as pl
from jax.experimental.pallas import tpu as pltpu
from jax.experimental.pallas import tpu_sc as plsc

N = 10000
E = 320000

NC = 2
NS = 16
CHUNK = 128
KI = 32
NBLK = 5
NCH = KI * NBLK
EP = NCH * CHUNK
EPAD = NS * EP
TRASH = N
RPT = 640
R = NS * RPT
NOUT = R


def _make_seg_sum(dh):
  mesh = plsc.VectorSubcoreMesh(core_axis_name="c", subcore_axis_name="s",
                                num_cores=NC, num_subcores=NS)

  @functools.partial(
      pl.kernel,
      out_type=jax.ShapeDtypeStruct((NC, NOUT, dh), jnp.float32),
      mesh=mesh,
      scratch_types=[
          pltpu.VMEM((KI, CHUNK), jnp.int32),
          pltpu.VMEM((KI, CHUNK), jnp.int32),
          pltpu.VMEM((CHUNK, dh), jnp.float32),
          pltpu.VMEM((CHUNK, dh), jnp.float32),
          pltpu.VMEM_SHARED((R, dh), jnp.float32),
          pltpu.SemaphoreType.DMA,
          pltpu.SemaphoreType.DMA,
      ],
  )
  def seg_sum(y_hbm, srcs_hbm, dsts_hbm, zeros_hbm, out_hbm,
              src_v, dst_v, rows0, rows1, acc_sh, sem0, sem1):
    c = lax.axis_index("c")
    s = lax.axis_index("s")
    pltpu.sync_copy(zeros_hbm, acc_sh.at[pl.ds(s * RPT, RPT)])
    plsc.subcore_barrier()
    table = y_hbm.at[c]

    def blk(bi, carry):
      pltpu.sync_copy(srcs_hbm.at[s].at[pl.ds(bi * KI, KI)], src_v)
      pltpu.sync_copy(dsts_hbm.at[s].at[pl.ds(bi * KI, KI)], dst_v)
      pltpu.async_copy(table.at[src_v.at[0]], rows0, sem0)

      def pair(k, carry2):
        j0 = 2 * k
        pltpu.make_async_copy(table.at[src_v.at[0]], rows0, sem0).wait()
        pltpu.async_copy(table.at[src_v.at[j0 + 1]], rows1, sem1)
        pltpu.sync_copy(rows0, acc_sh.at[dst_v.at[j0]], add=True)
        pltpu.make_async_copy(table.at[src_v.at[0]], rows1, sem1).wait()

        @pl.when(j0 + 2 < KI)
        def _():
          pltpu.async_copy(table.at[src_v.at[j0 + 2]], rows0, sem0)

        pltpu.sync_copy(rows1, acc_sh.at[dst_v.at[j0 + 1]], add=True)
        return carry2

      return lax.fori_loop(0, KI // 2, pair, carry)

    lax.fori_loop(0, NBLK, blk, 0)
    plsc.subcore_barrier()
    pltpu.sync_copy(acc_sh.at[pl.ds(s * RPT, RPT)],
                    out_hbm.at[c].at[pl.ds(s * RPT, RPT)])

  return seg_sum


_seg_sum_128 = _make_seg_sum(128)

KI2 = 40
NBLK2 = 2
NCH2 = KI2 * NBLK2

_fin_mesh = plsc.VectorSubcoreMesh(core_axis_name="c", subcore_axis_name="s",
                                   num_cores=NC, num_subcores=NS)


@functools.partial(
    pl.kernel,
    out_type=jax.ShapeDtypeStruct((NC, NOUT, 128), jnp.float32),
    mesh=_fin_mesh,
    scratch_types=[
        pltpu.VMEM((KI2, CHUNK), jnp.int32),
        pltpu.VMEM((KI2, CHUNK), jnp.int32),
        pltpu.VMEM((CHUNK, 128), jnp.float32),
        pltpu.VMEM((CHUNK, 128), jnp.float32),
        pltpu.VMEM_SHARED((R, 128), jnp.float32),
        pltpu.SemaphoreType.DMA,
        pltpu.SemaphoreType.DMA,
    ],
)
def _seg_sum_final(y_hbm, srcs_hbm, dsts_hbm, zeros_hbm, out_hbm,
                   src_v, dst_v, rows0, rows1, acc_sh, sem0, sem1):
  c = lax.axis_index("c")
  s = lax.axis_index("s")
  pltpu.sync_copy(zeros_hbm, acc_sh.at[pl.ds(s * RPT, RPT)])
  plsc.subcore_barrier()
  table = y_hbm.at[0]

  def blk(bi, carry):
    pltpu.sync_copy(srcs_hbm.at[c].at[s].at[pl.ds(bi * KI2, KI2)], src_v)
    pltpu.sync_copy(dsts_hbm.at[c].at[s].at[pl.ds(bi * KI2, KI2)], dst_v)
    pltpu.async_copy(table.at[src_v.at[0]], rows0, sem0)

    def pair(k, carry2):
      j0 = 2 * k
      pltpu.make_async_copy(table.at[src_v.at[0]], rows0, sem0).wait()
      pltpu.async_copy(table.at[src_v.at[j0 + 1]], rows1, sem1)
      pltpu.sync_copy(rows0, acc_sh.at[dst_v.at[j0]], add=True)
      pltpu.make_async_copy(table.at[src_v.at[0]], rows1, sem1).wait()

      @pl.when(j0 + 2 < KI2)
      def _():
        pltpu.async_copy(table.at[src_v.at[j0 + 2]], rows0, sem0)

      pltpu.sync_copy(rows1, acc_sh.at[dst_v.at[j0 + 1]], add=True)
      return carry2

    return lax.fori_loop(0, KI2 // 2, pair, carry)

  lax.fori_loop(0, NBLK2, blk, 0)
  plsc.subcore_barrier()
  pltpu.sync_copy(acc_sh.at[pl.ds(s * RPT, RPT)],
                  out_hbm.at[c].at[pl.ds(s * RPT, RPT)])

_deg_mesh = plsc.VectorSubcoreMesh(core_axis_name="c", subcore_axis_name="s",
                                   num_cores=NC, num_subcores=NS)


@functools.partial(
    pl.kernel,
    out_type=jax.ShapeDtypeStruct((NC, NOUT, 128), jnp.float32),
    mesh=_deg_mesh,
    scratch_types=[
        pltpu.VMEM((KI2, CHUNK), jnp.int32),
        pltpu.VMEM((CHUNK, 128), jnp.float32),
        pltpu.VMEM_SHARED((R, 128), jnp.float32),
        pltpu.SemaphoreType.DMA,
    ],
)
def _deg_kernel(dsts_hbm, ones_hbm, zeros_hbm, out_hbm,
                dst_v, rows_v, acc_sh, deg_sem):
  c = lax.axis_index("c")
  s = lax.axis_index("s")
  pltpu.sync_copy(ones_hbm, rows_v)
  pltpu.sync_copy(zeros_hbm, acc_sh.at[pl.ds(s * RPT, RPT)])
  plsc.subcore_barrier()

  def blk(bi, carry):
    pltpu.sync_copy(dsts_hbm.at[c].at[s].at[pl.ds(bi * KI2, KI2)], dst_v)

    def fire(j, carry2):
      pltpu.async_copy(rows_v, acc_sh.at[dst_v.at[j]], deg_sem, add=True)
      return carry2

    lax.fori_loop(0, KI2, fire, carry)

    def drain(j, carry2):
      pltpu.make_async_copy(rows_v, acc_sh.at[dst_v.at[0]], deg_sem).wait()
      return carry2

    return lax.fori_loop(0, KI2, drain, carry)

  lax.fori_loop(0, NBLK2, blk, 0)
  plsc.subcore_barrier()
  pltpu.sync_copy(acc_sh.at[pl.ds(s * RPT, RPT)],
                  out_hbm.at[lax.axis_index("c")].at[pl.ds(s * RPT, RPT)])


B = 1000
_GRID = N // B


def _pre0_body(deg_ref, x_ref, w_ref, dinv_ref, y_ref):
  dinv = lax.rsqrt(deg_ref[0, :, 0:1] + deg_ref[1, :, 0:1] + 1.0)
  xw = jnp.dot(x_ref[...], w_ref[...], preferred_element_type=jnp.float32)
  y = xw * dinv
  dh = y.shape[-1] // 2
  dinv_ref[...] = dinv
  y_ref[0] = y[:, :dh]
  y_ref[1] = y[:, dh:]


def _pre_body(dinv_ref, h_ref, w_ref, y_ref):
  xw = jnp.dot(h_ref[...], w_ref[...], preferred_element_type=jnp.float32)
  y = xw * dinv_ref[...]
  dh = y.shape[-1] // 2
  y_ref[0] = y[:, :dh]
  y_ref[1] = y[:, dh:]


def _pre3_body(dinv_ref, h_ref, w_ref, y_ref):
  xw = jnp.dot(h_ref[...], w_ref[...], preferred_element_type=jnp.float32)
  y_ref[0] = xw * dinv_ref[...]
  y_ref[1] = jnp.zeros_like(xw)


def _post_body(a_ref, y_ref, dinv_ref, res_ref, b_ref, g_ref, be_ref, o_ref):
  agg = jnp.concatenate([a_ref[0], a_ref[1]], axis=-1)
  y = jnp.concatenate([y_ref[0], y_ref[1]], axis=-1)
  dinv = dinv_ref[...]
  conv = dinv * (agg + y) + b_ref[...]
  mu = jnp.mean(conv, axis=-1, keepdims=True)
  var = jnp.mean((conv - mu) ** 2, axis=-1, keepdims=True)
  xn = (conv - mu) * lax.rsqrt(var + 1e-5) * g_ref[...] + be_ref[...]
  o = jnp.where(xn > 0, xn, jnp.exp(jnp.minimum(xn, 0.0)) - 1.0)
  o_ref[...] = o + res_ref[...]


def _post_pre_body(a_ref, y_ref, dinv_ref, res_ref, b_ref, g_ref, be_ref,
                   w_ref, h_ref, yn_ref):
  agg = jnp.concatenate([a_ref[0], a_ref[1]], axis=-1)
  y = jnp.concatenate([y_ref[0], y_ref[1]], axis=-1)
  dinv = dinv_ref[...]
  conv = dinv * (agg + y) + b_ref[...]
  mu = jnp.mean(conv, axis=-1, keepdims=True)
  var = jnp.mean((conv - mu) ** 2, axis=-1, keepdims=True)
  xn = (conv - mu) * lax.rsqrt(var + 1e-5) * g_ref[...] + be_ref[...]
  o = jnp.where(xn > 0, xn, jnp.exp(jnp.minimum(xn, 0.0)) - 1.0)
  h = o + res_ref[...]
  h_ref[...] = h
  xw = jnp.dot(h, w_ref[...], preferred_element_type=jnp.float32)
  yn = xw * dinv
  dh = yn.shape[-1] // 2
  yn_ref[0] = yn[:, :dh]
  yn_ref[1] = yn[:, dh:]


def _post_pre3_body(a_ref, y_ref, dinv_ref, res_ref, b_ref, g_ref, be_ref,
                    w_ref, h_ref, yn_ref):
  agg = jnp.concatenate([a_ref[0], a_ref[1]], axis=-1)
  y = jnp.concatenate([y_ref[0], y_ref[1]], axis=-1)
  dinv = dinv_ref[...]
  conv = dinv * (agg + y) + b_ref[...]
  mu = jnp.mean(conv, axis=-1, keepdims=True)
  var = jnp.mean((conv - mu) ** 2, axis=-1, keepdims=True)
  xn = (conv - mu) * lax.rsqrt(var + 1e-5) * g_ref[...] + be_ref[...]
  o = jnp.where(xn > 0, xn, jnp.exp(jnp.minimum(xn, 0.0)) - 1.0)
  h = o + res_ref[...]
  h_ref[...] = h
  xw = jnp.dot(h, w_ref[...], preferred_element_type=jnp.float32)
  yn_ref[0] = xw * dinv


def _final_body(a_ref, y_ref, dinv_ref, b_ref, o_ref):
  o_ref[...] = dinv_ref[...] * (a_ref[0] + a_ref[1] + y_ref[0]) + b_ref[...]


def _row_spec(d):
  return pl.BlockSpec((B, d), lambda i: (i, 0))


def _pair_spec(dh):
  return pl.BlockSpec((2, B, dh), lambda i: (0, i, 0))


def _full_spec(shape):
  return pl.BlockSpec(shape, lambda i: tuple(0 for _ in shape))


def _pre0_call(degcol, x, w):
  dout = w.shape[1]
  return pl.pallas_call(
      _pre0_body,
      grid=(_GRID,),
      in_specs=[pl.BlockSpec((2, B, 128), lambda i: (0, i, 0)),
                _row_spec(x.shape[1]), _full_spec(w.shape)],
      out_specs=[_row_spec(1), _pair_spec(dout // 2)],
      out_shape=[
          jax.ShapeDtypeStruct((N, 1), jnp.float32),
          jax.ShapeDtypeStruct((2, N, dout // 2), jnp.float32),
      ],
  )(degcol, x, w)


def _pre_call(dinv, h, w):
  dout = w.shape[1]
  return pl.pallas_call(
      _pre_body,
      grid=(_GRID,),
      in_specs=[_row_spec(1), _row_spec(h.shape[1]), _full_spec(w.shape)],
      out_specs=_pair_spec(dout // 2),
      out_shape=jax.ShapeDtypeStruct((2, N, dout // 2), jnp.float32),
  )(dinv, h, w)


def _post_call(agg, y, dinv, res, b, g, be):
  dh = agg.shape[-1]
  d = 2 * dh
  return pl.pallas_call(
      _post_body,
      grid=(_GRID,),
      in_specs=[_pair_spec(dh), _pair_spec(dh), _row_spec(1), _row_spec(d),
                _full_spec((1, d)), _full_spec((1, d)), _full_spec((1, d))],
      out_specs=_row_spec(d),
      out_shape=jax.ShapeDtypeStruct((N, d), jnp.float32),
  )(agg, y, dinv, res, b.reshape(1, d), g.reshape(1, d), be.reshape(1, d))


def _post_pre_call(agg, y, dinv, res, b, g, be, w, final):
  dh = agg.shape[-1]
  d = 2 * dh
  dout = w.shape[1]
  body = _post_pre3_body if final else _post_pre_body
  ydh = dout if final else dout // 2
  ny = 1 if final else 2
  return pl.pallas_call(
      body,
      grid=(_GRID,),
      in_specs=[_pair_spec(dh), _pair_spec(dh), _row_spec(1), _row_spec(d),
                _full_spec((1, d)), _full_spec((1, d)), _full_spec((1, d)),
                _full_spec(w.shape)],
      out_specs=[_row_spec(d), pl.BlockSpec((ny, B, ydh), lambda i: (0, i, 0))],
      out_shape=[
          jax.ShapeDtypeStruct((N, d), jnp.float32),
          jax.ShapeDtypeStruct((ny, N, ydh), jnp.float32),
      ],
  )(agg, y, dinv, res, b.reshape(1, d), g.reshape(1, d), be.reshape(1, d), w)


def _pre3_call(dinv, h, w):
  dout = w.shape[1]
  return pl.pallas_call(
      _pre3_body,
      grid=(_GRID,),
      in_specs=[_row_spec(1), _row_spec(h.shape[1]), _full_spec(w.shape)],
      out_specs=_pair_spec(dout),
      out_shape=jax.ShapeDtypeStruct((2, N, dout), jnp.float32),
  )(dinv, h, w)


def _final_call(agg, y, dinv, b):
  d = agg.shape[-1]
  return pl.pallas_call(
      _final_body,
      grid=(_GRID,),
      in_specs=[_pair_spec(d), pl.BlockSpec((1, B, d), lambda i: (0, i, 0)),
                _row_spec(1), _full_spec((1, d))],
      out_specs=_row_spec(d),
      out_shape=jax.ShapeDtypeStruct((N, d), jnp.float32),
  )(agg, y, dinv, b.reshape(1, d))


def kernel(x, edge_index, W0, b0, W1, b1, W2, b2, W3, b3,
           g0, be0, g1, be1, g2, be2):
  src = edge_index[0].astype(jnp.int32)
  dst = edge_index[1].astype(jnp.int32)
  pad = EPAD - E
  src_p = jnp.pad(src, (0, pad))
  dst_p = jnp.pad(dst, (0, pad), constant_values=TRASH)
  srcs = src_p.reshape(NS, NCH, CHUNK)
  dsts = dst_p.reshape(NS, NCH, CHUNK)
  srcs2 = src_p.reshape(NC, NS, NCH2, CHUNK)
  dsts2 = dst_p.reshape(NC, NS, NCH2, CHUNK)
  zeros128 = jnp.zeros((RPT, 128), jnp.float32)
  ones128 = jnp.ones((CHUNK, 128), jnp.float32)

  deg_out = _deg_kernel(dsts2, ones128, zeros128)

  dinv, y = _pre0_call(deg_out, x, W0)
  agg = _seg_sum_128(y, srcs, dsts, zeros128)
  h, y = _post_pre_call(agg, y, dinv, jnp.zeros((N, 256), jnp.float32),
                        b0, g0, be0, W1, final=False)
  agg = _seg_sum_128(y, srcs, dsts, zeros128)
  h, y = _post_pre_call(agg, y, dinv, h, b1, g1, be1, W2, final=False)
  agg = _seg_sum_128(y, srcs, dsts, zeros128)
  h, y = _post_pre_call(agg, y, dinv, h, b2, g2, be2, W3, final=True)
  agg = _seg_sum_final(y, srcs2, dsts2, zeros128)
  return _final_call(agg, y, dinv, b3)
